# Initial kernel scaffold; baseline (speedup 1.0000x reference)
import jax
import jax.numpy as jnp
from jax.experimental import pallas as pl


def kernel(x, edge_index, W_l1, b_l1, W_r1, W_l2, b_l2, W_r2):
    raise NotImplementedError("write your pallas kernel here")


# speedup vs baseline: 6.6562x; 6.6562x over previous
import functools

import jax
import jax.numpy as jnp
from jax import lax
from jax.experimental import pallas as pl
from jax.experimental.pallas import tpu as pltpu
from jax.experimental.pallas import tpu_sc as plsc

N = 10000
E = 320000
D = 128
NC = 2
NS = 16
NW = NC * NS
EPW = E // NW
B = 80
SEC = 25
NSEC = 5
NACC = 10240
RPT = NACC // NS

_SC_PARAMS = dict(
    mesh=plsc.VectorSubcoreMesh(core_axis_name="c", subcore_axis_name="s"),
    compiler_params=pltpu.CompilerParams(needs_layout_passes=False),
)


def _sc_agg_body(y_hbm, src_hbm, dst_hbm, z_hbm,
                 agg_out,
                 src_v, dst_v, rows_v, acc, sem):
    c = lax.axis_index("c")
    s = lax.axis_index("s")
    wid = c * NS + s
    row0 = s * RPT
    pltpu.sync_copy(z_hbm, rows_v)

    def z_body(k, carry):
        pltpu.sync_copy(rows_v, acc.at[pl.ds(row0 + k * B, B)])
        return carry

    lax.fori_loop(0, RPT // B, z_body, 0)
    plsc.subcore_barrier()

    def sec_body(g, carry):
        pltpu.sync_copy(src_hbm.at[wid, g], src_v)
        pltpu.sync_copy(dst_hbm.at[wid, g], dst_v)

        def body(j, carry2):
            pltpu.async_copy(y_hbm.at[src_v.at[j]], rows_v, sem).wait()
            pltpu.sync_copy(rows_v, acc.at[dst_v.at[j]], add=True)
            return carry2

        return lax.fori_loop(0, SEC, body, carry)

    lax.fori_loop(0, NSEC, sec_body, 0)
    plsc.subcore_barrier()

    def d_body(k, carry):
        pltpu.sync_copy(acc.at[pl.ds(row0 + k * B, B)], rows_v)
        pltpu.sync_copy(rows_v, agg_out.at[c, s, pl.ds(k * B, B)])
        return carry

    lax.fori_loop(0, RPT // B, d_body, 0)


_sc_agg = functools.partial(
    pl.kernel,
    out_type=jax.ShapeDtypeStruct((NC, NS, RPT, D), jnp.float32),
    scratch_types=[
        pltpu.VMEM((SEC, B), jnp.int32),
        pltpu.VMEM((SEC, B), jnp.int32),
        pltpu.VMEM((B, D), jnp.float32),
        pltpu.VMEM_SHARED((NACC, D), jnp.float32),
        pltpu.SemaphoreType.DMA,
    ],
    **_SC_PARAMS,
)(_sc_agg_body)


def _sc_deg_body(dst_hbm, z_hbm, one_hbm,
                 deg_out,
                 dst_v, rows_v, acc, sem):
    c = lax.axis_index("c")
    s = lax.axis_index("s")
    wid = c * NS + s
    row0 = s * RPT
    pltpu.sync_copy(z_hbm, rows_v)

    def z_body(k, carry):
        pltpu.sync_copy(rows_v, acc.at[pl.ds(row0 + k * B, B)])
        return carry

    lax.fori_loop(0, RPT // B, z_body, 0)
    pltpu.sync_copy(one_hbm, rows_v)
    plsc.subcore_barrier()

    def sec_body(g, carry):
        pltpu.sync_copy(dst_hbm.at[wid, g], dst_v)

        def body(j, carry2):
            pltpu.sync_copy(rows_v, acc.at[dst_v.at[j]], add=True)
            return carry2

        return lax.fori_loop(0, SEC, body, carry)

    lax.fori_loop(0, NSEC, sec_body, 0)
    plsc.subcore_barrier()

    def d_body(k, carry):
        pltpu.sync_copy(acc.at[pl.ds(row0 + k * B, B)], rows_v)
        pltpu.sync_copy(rows_v, deg_out.at[c, s, pl.ds(k * B, B)])
        return carry

    lax.fori_loop(0, RPT // B, d_body, 0)


_sc_deg = functools.partial(
    pl.kernel,
    out_type=jax.ShapeDtypeStruct((NC, NS, RPT, D), jnp.float32),
    scratch_types=[
        pltpu.VMEM((SEC, B), jnp.int32),
        pltpu.VMEM((B, D), jnp.float32),
        pltpu.VMEM_SHARED((NACC, D), jnp.float32),
        pltpu.SemaphoreType.DMA,
    ],
    **_SC_PARAMS,
)(_sc_deg_body)


BLK = 2000


def _tc_pre_body(x_ref, wl_ref, wr_ref, b_ref, y_ref, z_ref):
    xb = x_ref[...]
    y_ref[...] = jnp.dot(xb, wl_ref[...], precision=lax.Precision.HIGHEST)
    z_ref[...] = jnp.dot(xb, wr_ref[...], precision=lax.Precision.HIGHEST) + b_ref[...]


def _tc_pre(x, wlT, wrT, b):
    return pl.pallas_call(
        _tc_pre_body,
        grid=(N // BLK,),
        in_specs=[
            pl.BlockSpec((BLK, D), lambda i: (i, 0)),
            pl.BlockSpec((D, D), lambda i: (0, 0)),
            pl.BlockSpec((D, D), lambda i: (0, 0)),
            pl.BlockSpec((1, D), lambda i: (0, 0)),
        ],
        out_specs=[
            pl.BlockSpec((BLK, D), lambda i: (i, 0)),
            pl.BlockSpec((BLK, D), lambda i: (i, 0)),
        ],
        out_shape=[
            jax.ShapeDtypeStruct((N, D), jnp.float32),
            jax.ShapeDtypeStruct((N, D), jnp.float32),
        ],
    )(x, wlT, wrT, b)


def _tc_mid_body(p_ref, dg_ref, z1_ref, wl_ref, wr_ref, b_ref, y_ref, z_ref):
    deg = dg_ref[0, :, 0:1] + dg_ref[1, :, 0:1]
    inv = 1.0 / jnp.maximum(deg, 1.0)
    h = (p_ref[0] + p_ref[1]) * inv + z1_ref[...]
    h = jnp.maximum(h, 0.0)
    y_ref[...] = jnp.dot(h, wl_ref[...], precision=lax.Precision.HIGHEST)
    z_ref[...] = jnp.dot(h, wr_ref[...], precision=lax.Precision.HIGHEST) + b_ref[...]


def _tc_mid(p, dg, z1, wlT, wrT, b):
    return pl.pallas_call(
        _tc_mid_body,
        grid=(N // BLK,),
        in_specs=[
            pl.BlockSpec((NC, BLK, D), lambda i: (0, i, 0)),
            pl.BlockSpec((NC, BLK, D), lambda i: (0, i, 0)),
            pl.BlockSpec((BLK, D), lambda i: (i, 0)),
            pl.BlockSpec((D, D), lambda i: (0, 0)),
            pl.BlockSpec((D, D), lambda i: (0, 0)),
            pl.BlockSpec((1, D), lambda i: (0, 0)),
        ],
        out_specs=[
            pl.BlockSpec((BLK, D), lambda i: (i, 0)),
            pl.BlockSpec((BLK, D), lambda i: (i, 0)),
        ],
        out_shape=[
            jax.ShapeDtypeStruct((N, D), jnp.float32),
            jax.ShapeDtypeStruct((N, D), jnp.float32),
        ],
    )(p, dg, z1, wlT, wrT, b)


def _tc_post_body(p_ref, dg_ref, z2_ref, o_ref):
    deg = dg_ref[0, :, 0:1] + dg_ref[1, :, 0:1]
    inv = 1.0 / jnp.maximum(deg, 1.0)
    o_ref[...] = (p_ref[0] + p_ref[1]) * inv + z2_ref[...]


def _tc_post(p, dg, z2):
    return pl.pallas_call(
        _tc_post_body,
        grid=(N // BLK,),
        in_specs=[
            pl.BlockSpec((NC, BLK, D), lambda i: (0, i, 0)),
            pl.BlockSpec((NC, BLK, D), lambda i: (0, i, 0)),
            pl.BlockSpec((BLK, D), lambda i: (i, 0)),
        ],
        out_specs=pl.BlockSpec((BLK, D), lambda i: (i, 0)),
        out_shape=jax.ShapeDtypeStruct((N, D), jnp.float32),
    )(p, dg, z2)


def kernel(x, edge_index, W_l1, b_l1, W_r1, W_l2, b_l2, W_r2):
    src = edge_index[0].astype(jnp.int32).reshape(NW, NSEC, SEC, B)
    dst = edge_index[1].astype(jnp.int32).reshape(NW, NSEC, SEC, B)
    z = jnp.zeros((B, D), jnp.float32)
    ones = jnp.ones((B, D), jnp.float32)

    dg = _sc_deg(dst, z, ones).reshape(NC, NACC, D)
    y1, z1 = _tc_pre(x, W_l1.T, W_r1.T, b_l1.reshape(1, D))
    p1 = _sc_agg(y1, src, dst, z).reshape(NC, NACC, D)
    y2, z2 = _tc_mid(p1, dg, z1, W_l2.T, W_r2.T, b_l2.reshape(1, D))
    p2 = _sc_agg(y2, src, dst, z).reshape(NC, NACC, D)
    return _tc_post(p2, dg, z2)


# speedup vs baseline: 8.0501x; 1.2094x over previous
import functools

import jax
import jax.numpy as jnp
from jax import lax
from jax.experimental import pallas as pl
from jax.experimental.pallas import tpu as pltpu
from jax.experimental.pallas import tpu_sc as plsc

N = 10000
E = 320000
D = 128
NC = 2
NS = 16
NW = NC * NS
EPW = E // NW
B = 80
SEC = 25
NSEC = 5
CHUNKS = SEC * NSEC
NACC = 10240
RPT = NACC // NS

_SC_PARAMS = dict(
    mesh=plsc.VectorSubcoreMesh(core_axis_name="c", subcore_axis_name="s"),
    compiler_params=pltpu.CompilerParams(needs_layout_passes=False),
)


def _sc_agg_body(y_hbm, src_hbm, dst_hbm, z_hbm,
                 agg_out,
                 src_v, dst_v, rows_v, acc, sem):
    c = lax.axis_index("c")
    s = lax.axis_index("s")
    wid = c * NS + s
    row0 = s * RPT
    pltpu.sync_copy(z_hbm, rows_v.at[0])

    def z_body(k, carry):
        pltpu.sync_copy(rows_v.at[0], acc.at[pl.ds(row0 + k * B, B)])
        return carry

    lax.fori_loop(0, RPT // B, z_body, 0)
    plsc.subcore_barrier()

    pltpu.sync_copy(src_hbm.at[wid, 0], src_v.at[0])
    pltpu.sync_copy(dst_hbm.at[wid, 0], dst_v.at[0])
    pltpu.async_copy(y_hbm.at[src_v.at[0, 0]], rows_v.at[0], sem)

    def body(j, carry):
        sec = j // SEC
        off = j % SEC
        par = sec % 2
        buf = j % 2
        pltpu.make_async_copy(
            y_hbm.at[src_v.at[par, off]], rows_v.at[buf], sem).wait()

        jn = j + 1

        @pl.when(jn < CHUNKS)
        def _prefetch():
            secn = jn // SEC
            offn = jn % SEC
            parn = secn % 2

            @pl.when(offn == 0)
            def _refill():
                pltpu.sync_copy(src_hbm.at[wid, secn], src_v.at[parn])
                pltpu.sync_copy(dst_hbm.at[wid, secn], dst_v.at[parn])

            pltpu.async_copy(
                y_hbm.at[src_v.at[parn, offn]], rows_v.at[jn % 2], sem)

        pltpu.sync_copy(rows_v.at[buf], acc.at[dst_v.at[par, off]], add=True)
        return carry

    lax.fori_loop(0, CHUNKS, body, 0)
    plsc.subcore_barrier()

    def d_body(k, carry):
        pltpu.sync_copy(acc.at[pl.ds(row0 + k * B, B)], rows_v.at[0])
        pltpu.sync_copy(rows_v.at[0], agg_out.at[c, s, pl.ds(k * B, B)])
        return carry

    lax.fori_loop(0, RPT // B, d_body, 0)


_sc_agg = functools.partial(
    pl.kernel,
    out_type=jax.ShapeDtypeStruct((NC, NS, RPT, D), jnp.float32),
    scratch_types=[
        pltpu.VMEM((2, SEC, B), jnp.int32),
        pltpu.VMEM((2, SEC, B), jnp.int32),
        pltpu.VMEM((2, B, D), jnp.float32),
        pltpu.VMEM_SHARED((NACC, D), jnp.float32),
        pltpu.SemaphoreType.DMA,
    ],
    **_SC_PARAMS,
)(_sc_agg_body)


def _sc_deg_body(dst_hbm, z_hbm, one_hbm,
                 deg_out,
                 dst_v, rows_v, acc, sem):
    c = lax.axis_index("c")
    s = lax.axis_index("s")
    wid = c * NS + s
    row0 = s * RPT
    pltpu.sync_copy(z_hbm, rows_v)

    def z_body(k, carry):
        pltpu.sync_copy(rows_v, acc.at[pl.ds(row0 + k * B, B)])
        return carry

    lax.fori_loop(0, RPT // B, z_body, 0)
    pltpu.sync_copy(one_hbm, rows_v)
    plsc.subcore_barrier()

    def sec_body(g, carry):
        pltpu.sync_copy(dst_hbm.at[wid, g], dst_v)

        def body(j, carry2):
            pltpu.sync_copy(rows_v, acc.at[dst_v.at[j]], add=True)
            return carry2

        return lax.fori_loop(0, SEC, body, carry)

    lax.fori_loop(0, NSEC, sec_body, 0)
    plsc.subcore_barrier()

    def d_body(k, carry):
        pltpu.sync_copy(acc.at[pl.ds(row0 + k * B, B)], rows_v)
        pltpu.sync_copy(rows_v, deg_out.at[c, s, pl.ds(k * B, B)])
        return carry

    lax.fori_loop(0, RPT // B, d_body, 0)


_sc_deg = functools.partial(
    pl.kernel,
    out_type=jax.ShapeDtypeStruct((NC, NS, RPT, D), jnp.float32),
    scratch_types=[
        pltpu.VMEM((SEC, B), jnp.int32),
        pltpu.VMEM((B, D), jnp.float32),
        pltpu.VMEM_SHARED((NACC, D), jnp.float32),
        pltpu.SemaphoreType.DMA,
    ],
    **_SC_PARAMS,
)(_sc_deg_body)


BLK = 2000


def _tc_pre_body(x_ref, wl_ref, wr_ref, b_ref, y_ref, z_ref):
    xb = x_ref[...]
    y_ref[...] = jnp.dot(xb, wl_ref[...], precision=lax.Precision.HIGHEST)
    z_ref[...] = jnp.dot(xb, wr_ref[...], precision=lax.Precision.HIGHEST) + b_ref[...]


def _tc_pre(x, wlT, wrT, b):
    return pl.pallas_call(
        _tc_pre_body,
        grid=(N // BLK,),
        in_specs=[
            pl.BlockSpec((BLK, D), lambda i: (i, 0)),
            pl.BlockSpec((D, D), lambda i: (0, 0)),
            pl.BlockSpec((D, D), lambda i: (0, 0)),
            pl.BlockSpec((1, D), lambda i: (0, 0)),
        ],
        out_specs=[
            pl.BlockSpec((BLK, D), lambda i: (i, 0)),
            pl.BlockSpec((BLK, D), lambda i: (i, 0)),
        ],
        out_shape=[
            jax.ShapeDtypeStruct((N, D), jnp.float32),
            jax.ShapeDtypeStruct((N, D), jnp.float32),
        ],
    )(x, wlT, wrT, b)


def _tc_mid_body(p_ref, dg_ref, z1_ref, wl_ref, wr_ref, b_ref, y_ref, z_ref):
    deg = dg_ref[0, :, 0:1] + dg_ref[1, :, 0:1]
    inv = 1.0 / jnp.maximum(deg, 1.0)
    h = (p_ref[0] + p_ref[1]) * inv + z1_ref[...]
    h = jnp.maximum(h, 0.0)
    y_ref[...] = jnp.dot(h, wl_ref[...], precision=lax.Precision.HIGHEST)
    z_ref[...] = jnp.dot(h, wr_ref[...], precision=lax.Precision.HIGHEST) + b_ref[...]


def _tc_mid(p, dg, z1, wlT, wrT, b):
    return pl.pallas_call(
        _tc_mid_body,
        grid=(N // BLK,),
        in_specs=[
            pl.BlockSpec((NC, BLK, D), lambda i: (0, i, 0)),
            pl.BlockSpec((NC, BLK, D), lambda i: (0, i, 0)),
            pl.BlockSpec((BLK, D), lambda i: (i, 0)),
            pl.BlockSpec((D, D), lambda i: (0, 0)),
            pl.BlockSpec((D, D), lambda i: (0, 0)),
            pl.BlockSpec((1, D), lambda i: (0, 0)),
        ],
        out_specs=[
            pl.BlockSpec((BLK, D), lambda i: (i, 0)),
            pl.BlockSpec((BLK, D), lambda i: (i, 0)),
        ],
        out_shape=[
            jax.ShapeDtypeStruct((N, D), jnp.float32),
            jax.ShapeDtypeStruct((N, D), jnp.float32),
        ],
    )(p, dg, z1, wlT, wrT, b)


def _tc_post_body(p_ref, dg_ref, z2_ref, o_ref):
    deg = dg_ref[0, :, 0:1] + dg_ref[1, :, 0:1]
    inv = 1.0 / jnp.maximum(deg, 1.0)
    o_ref[...] = (p_ref[0] + p_ref[1]) * inv + z2_ref[...]


def _tc_post(p, dg, z2):
    return pl.pallas_call(
        _tc_post_body,
        grid=(N // BLK,),
        in_specs=[
            pl.BlockSpec((NC, BLK, D), lambda i: (0, i, 0)),
            pl.BlockSpec((NC, BLK, D), lambda i: (0, i, 0)),
            pl.BlockSpec((BLK, D), lambda i: (i, 0)),
        ],
        out_specs=pl.BlockSpec((BLK, D), lambda i: (i, 0)),
        out_shape=jax.ShapeDtypeStruct((N, D), jnp.float32),
    )(p, dg, z2)


def kernel(x, edge_index, W_l1, b_l1, W_r1, W_l2, b_l2, W_r2):
    src = edge_index[0].astype(jnp.int32).reshape(NW, NSEC, SEC, B)
    dst = edge_index[1].astype(jnp.int32).reshape(NW, NSEC, SEC, B)
    z = jnp.zeros((B, D), jnp.float32)
    ones = jnp.ones((B, D), jnp.float32)

    dg = _sc_deg(dst, z, ones).reshape(NC, NACC, D)
    y1, z1 = _tc_pre(x, W_l1.T, W_r1.T, b_l1.reshape(1, D))
    p1 = _sc_agg(y1, src, dst, z).reshape(NC, NACC, D)
    y2, z2 = _tc_mid(p1, dg, z1, W_l2.T, W_r2.T, b_l2.reshape(1, D))
    p2 = _sc_agg(y2, src, dst, z).reshape(NC, NACC, D)
    return _tc_post(p2, dg, z2)


# speedup vs baseline: 9.6516x; 1.1989x over previous
import functools

import jax
import jax.numpy as jnp
from jax import lax
from jax.experimental import pallas as pl
from jax.experimental.pallas import tpu as pltpu
from jax.experimental.pallas import tpu_sc as plsc

N = 10000
E = 320000
D = 128
NC = 2
NS = 16
NW = NC * NS
EPW = E // NW
B = 80
SEC = 25
NSEC = 5
CHUNKS = SEC * NSEC
NACC = 10240
RPT = NACC // NS

_SC_PARAMS = dict(
    mesh=plsc.VectorSubcoreMesh(core_axis_name="c", subcore_axis_name="s"),
    compiler_params=pltpu.CompilerParams(needs_layout_passes=False),
)


def _sc_agg_body(y_hbm, src_hbm, dst_hbm, z_hbm,
                 agg_out,
                 src_v, dst_v, rows_v, acc, sem):
    c = lax.axis_index("c")
    s = lax.axis_index("s")
    wid = c * NS + s
    row0 = s * RPT
    pltpu.sync_copy(z_hbm, rows_v.at[0])

    def z_body(k, carry):
        pltpu.sync_copy(rows_v.at[0], acc.at[pl.ds(row0 + k * B, B)])
        return carry

    lax.fori_loop(0, RPT // B, z_body, 0)
    plsc.subcore_barrier()

    pltpu.sync_copy(src_hbm.at[wid, 0], src_v.at[0])
    pltpu.sync_copy(dst_hbm.at[wid, 0], dst_v.at[0])
    pltpu.async_copy(y_hbm.at[src_v.at[0, 0]], rows_v.at[0], sem)

    def body(j, carry):
        sec = j // SEC
        off = j % SEC
        par = sec % 2
        buf = j % 2
        pltpu.make_async_copy(
            y_hbm.at[src_v.at[par, off]], rows_v.at[buf], sem).wait()

        jn = j + 1

        @pl.when(jn < CHUNKS)
        def _prefetch():
            secn = jn // SEC
            offn = jn % SEC
            parn = secn % 2

            @pl.when(offn == 0)
            def _refill():
                pltpu.sync_copy(src_hbm.at[wid, secn], src_v.at[parn])
                pltpu.sync_copy(dst_hbm.at[wid, secn], dst_v.at[parn])

            pltpu.async_copy(
                y_hbm.at[src_v.at[parn, offn]], rows_v.at[jn % 2], sem)

        pltpu.sync_copy(rows_v.at[buf], acc.at[dst_v.at[par, off]], add=True)
        return carry

    lax.fori_loop(0, CHUNKS, body, 0)
    plsc.subcore_barrier()

    def d_body(k, carry):
        pltpu.sync_copy(acc.at[pl.ds(row0 + k * B, B)], rows_v.at[0])
        pltpu.sync_copy(rows_v.at[0], agg_out.at[c, s, pl.ds(k * B, B)])
        return carry

    lax.fori_loop(0, RPT // B, d_body, 0)


_sc_agg = functools.partial(
    pl.kernel,
    out_type=jax.ShapeDtypeStruct((NC, NS, RPT, D), jnp.float32),
    scratch_types=[
        pltpu.VMEM((2, SEC, B), jnp.int32),
        pltpu.VMEM((2, SEC, B), jnp.int32),
        pltpu.VMEM((2, B, D), jnp.float32),
        pltpu.VMEM_SHARED((NACC, D), jnp.float32),
        pltpu.SemaphoreType.DMA,
    ],
    **_SC_PARAMS,
)(_sc_agg_body)


def _sc_deg_body(dst_hbm, zd_hbm, deg_out, dst_v, deg_v):
    c = lax.axis_index("c")
    s = lax.axis_index("s")
    wid = c * NS + s
    pltpu.sync_copy(zd_hbm, deg_v)
    ones16 = jnp.full((16,), 1.0, jnp.float32)

    def sec_body(g, carry):
        pltpu.sync_copy(dst_hbm.at[wid, g], dst_v)

        def body(j, carry2):
            def m_body(m, carry3):
                idx = dst_v[j, pl.ds(m * 16, 16)]
                plsc.addupdate_scatter(deg_v, [idx], ones16)
                return carry3

            return lax.fori_loop(0, B // 16, m_body, carry2)

        return lax.fori_loop(0, SEC, body, carry)

    lax.fori_loop(0, NSEC, sec_body, 0)
    pltpu.sync_copy(deg_v, deg_out.at[wid])


_sc_deg = functools.partial(
    pl.kernel,
    out_type=jax.ShapeDtypeStruct((NW, NACC), jnp.float32),
    scratch_types=[
        pltpu.VMEM((SEC, B), jnp.int32),
        pltpu.VMEM((NACC,), jnp.float32),
    ],
    **_SC_PARAMS,
)(_sc_deg_body)


BLK = 2000


def _tc_pre_body(x_ref, wl_ref, wr_ref, b_ref, y_ref, z_ref):
    xb = x_ref[...]
    y_ref[...] = jnp.dot(xb, wl_ref[...], precision=lax.Precision.HIGHEST)
    z_ref[...] = jnp.dot(xb, wr_ref[...], precision=lax.Precision.HIGHEST) + b_ref[...]


def _tc_pre(x, wlT, wrT, b):
    return pl.pallas_call(
        _tc_pre_body,
        grid=(N // BLK,),
        in_specs=[
            pl.BlockSpec((BLK, D), lambda i: (i, 0)),
            pl.BlockSpec((D, D), lambda i: (0, 0)),
            pl.BlockSpec((D, D), lambda i: (0, 0)),
            pl.BlockSpec((1, D), lambda i: (0, 0)),
        ],
        out_specs=[
            pl.BlockSpec((BLK, D), lambda i: (i, 0)),
            pl.BlockSpec((BLK, D), lambda i: (i, 0)),
        ],
        out_shape=[
            jax.ShapeDtypeStruct((N, D), jnp.float32),
            jax.ShapeDtypeStruct((N, D), jnp.float32),
        ],
    )(x, wlT, wrT, b)


def _tc_mid_body(p_ref, dg_ref, z1_ref, wl_ref, wr_ref, b_ref, y_ref, z_ref):
    deg = jnp.sum(dg_ref[...], axis=1, keepdims=True)
    inv = 1.0 / jnp.maximum(deg, 1.0)
    h = (p_ref[0] + p_ref[1]) * inv + z1_ref[...]
    h = jnp.maximum(h, 0.0)
    y_ref[...] = jnp.dot(h, wl_ref[...], precision=lax.Precision.HIGHEST)
    z_ref[...] = jnp.dot(h, wr_ref[...], precision=lax.Precision.HIGHEST) + b_ref[...]


def _tc_mid(p, dg, z1, wlT, wrT, b):
    return pl.pallas_call(
        _tc_mid_body,
        grid=(N // BLK,),
        in_specs=[
            pl.BlockSpec((NC, BLK, D), lambda i: (0, i, 0)),
            pl.BlockSpec((BLK, NW), lambda i: (i, 0)),
            pl.BlockSpec((BLK, D), lambda i: (i, 0)),
            pl.BlockSpec((D, D), lambda i: (0, 0)),
            pl.BlockSpec((D, D), lambda i: (0, 0)),
            pl.BlockSpec((1, D), lambda i: (0, 0)),
        ],
        out_specs=[
            pl.BlockSpec((BLK, D), lambda i: (i, 0)),
            pl.BlockSpec((BLK, D), lambda i: (i, 0)),
        ],
        out_shape=[
            jax.ShapeDtypeStruct((N, D), jnp.float32),
            jax.ShapeDtypeStruct((N, D), jnp.float32),
        ],
    )(p, dg, z1, wlT, wrT, b)


def _tc_post_body(p_ref, dg_ref, z2_ref, o_ref):
    deg = jnp.sum(dg_ref[...], axis=1, keepdims=True)
    inv = 1.0 / jnp.maximum(deg, 1.0)
    o_ref[...] = (p_ref[0] + p_ref[1]) * inv + z2_ref[...]


def _tc_post(p, dg, z2):
    return pl.pallas_call(
        _tc_post_body,
        grid=(N // BLK,),
        in_specs=[
            pl.BlockSpec((NC, BLK, D), lambda i: (0, i, 0)),
            pl.BlockSpec((BLK, NW), lambda i: (i, 0)),
            pl.BlockSpec((BLK, D), lambda i: (i, 0)),
        ],
        out_specs=pl.BlockSpec((BLK, D), lambda i: (i, 0)),
        out_shape=jax.ShapeDtypeStruct((N, D), jnp.float32),
    )(p, dg, z2)


def kernel(x, edge_index, W_l1, b_l1, W_r1, W_l2, b_l2, W_r2):
    src = edge_index[0].astype(jnp.int32).reshape(NW, NSEC, SEC, B)
    dst = edge_index[1].astype(jnp.int32).reshape(NW, NSEC, SEC, B)
    z = jnp.zeros((B, D), jnp.float32)
    zd = jnp.zeros((NACC,), jnp.float32)

    dgT = _sc_deg(dst, zd).T
    y1, z1 = _tc_pre(x, W_l1.T, W_r1.T, b_l1.reshape(1, D))
    p1 = _sc_agg(y1, src, dst, z).reshape(NC, NACC, D)
    y2, z2 = _tc_mid(p1, dgT, z1, W_l2.T, W_r2.T, b_l2.reshape(1, D))
    p2 = _sc_agg(y2, src, dst, z).reshape(NC, NACC, D)
    return _tc_post(p2, dgT, z2)
